# baseline (speedup 1.0000x reference)
import functools

import jax
import jax.numpy as jnp
from jax import lax
from jax.experimental import pallas as pl
from jax.experimental.pallas import tpu as pltpu
from jax.experimental.pallas import tpu_sc as plsc

B, S, V, NOUNS = 128, 50, 9487, 1000
N = B * S
L = 16
NC, NS = 2, 16
NW = NC * NS
TCH = 200
RU = S * (NOUNS // TCH)
RIT = -(-RU // NW)
BCH = B // L


def _ge0(x):
    return 1 - lax.shift_right_logical(x, 31)


def _sc_body(inp_hbm, rel_hbm, t_hbm, m_hbm, out_hbm,
             t_v, m_v, bufr0, bufr1, stg0, stg1, out_v,
             semr0, semr1, semi0, semi1):
    wid = lax.axis_index("c") * NS + lax.axis_index("s")
    iota = lax.iota(jnp.int32, L)
    bufsr = (bufr0, bufr1)
    semsr = (semr0, semr1)

    def unit_su(u):
        u_cl = jnp.minimum(u, RU - 1)
        s = u_cl // 5
        t0 = pl.multiple_of((u_cl % 5) * TCH, 8)
        return s, t0, _ge0(RU - 1 - u).astype(jnp.float32)

    def start_r(u, slot):
        s, t0, _ = unit_su(u)
        return pltpu.async_copy(rel_hbm.at[s, pl.ds(t0, TCH), :],
                                bufsr[slot], semsr[slot])

    cpsr = [start_r(wid, 0), None]
    cpsr[1] = start_r(wid + NW, 1)
    pltpu.sync_copy(t_hbm, t_v)
    pltpu.sync_copy(m_hbm, m_v)

    lm_cps = []
    for srow_off, stg, sem in ((0, stg0, semi0), (NW, stg1, semi1)):
        s = jnp.minimum(wid + srow_off, S - 1)
        for k in range(BCH):
            t16 = jnp.clip(t_v[s, pl.ds(k * L, L)], 0, V - 1)
            lm_cps.append(pltpu.async_copy(
                inp_hbm.at[s].at[t16], stg.at[pl.ds(k * L, L), :], sem))

    m_acc = jnp.zeros((L,), jnp.float32)
    nm_acc = jnp.zeros((L,), jnp.float32)
    for srow_off in (0, NW):
        srow = wid + srow_off
        s = jnp.minimum(srow, S - 1)
        wg = _ge0(S - 1 - srow).astype(jnp.float32)
        for k in range(BCH):
            t16 = t_v[s, pl.ds(k * L, L)]
            m16 = m_v[s, pl.ds(k * L, L)]
            nm_acc += _ge0(NOUNS - t16).astype(jnp.float32) * wg
            m_acc += m16 * wg

    ss_acc = jnp.zeros((L,), jnp.float32)
    g_acc = jnp.zeros((L,), jnp.float32)
    for i in range(RIT):
        slot = i % 2
        u = wid + NW * i
        cpsr[slot].wait()
        buf = bufsr[slot]
        s, t0, wu = unit_su(u)

        def row(r, a, buf=buf):
            a0, a1, a2, a3 = a
            for rr in range(8):
                for k in range(0, BCH, 4):
                    x0 = buf[8 * r + rr, pl.ds(k * L, L)]
                    x1 = buf[8 * r + rr, pl.ds((k + 1) * L, L)]
                    x2 = buf[8 * r + rr, pl.ds((k + 2) * L, L)]
                    x3 = buf[8 * r + rr, pl.ds((k + 3) * L, L)]
                    a0 += x0 * x0
                    a1 += x1 * x1
                    a2 += x2 * x2
                    a3 += x3 * x3
            return (a0, a1, a2, a3)

        z = jnp.zeros((L,), jnp.float32)
        u0, u1, u2, u3 = lax.fori_loop(0, TCH // 8, row, (z, z, z, z))
        ss_acc += (u0 + u1 + u2 + u3) * wu

        for k in range(BCH):
            t16 = t_v[s, pl.ds(k * L, L)]
            tg = t16 - 1
            inr = (_ge0(tg - t0) * _ge0(t0 + (TCH - 1) - tg)).astype(jnp.float32)
            idx_t = jnp.clip(tg - t0, 0, TCH - 1)
            g_acc += plsc.load_gather(buf, [idx_t, iota + k * L]) * (inr * wu)

        if i + 2 < RIT:
            cpsr[slot] = start_r(wid + NW * (i + 2), slot)

    for cp in lm_cps:
        cp.wait()
    lm_acc = jnp.zeros((L,), jnp.float32)
    for srow_off, stg in ((0, stg0), (NW, stg1)):
        srow = wid + srow_off
        s = jnp.minimum(srow, S - 1)
        wg = _ge0(S - 1 - srow).astype(jnp.float32)
        for k in range(BCH):
            m16 = m_v[s, pl.ds(k * L, L)]
            d16 = plsc.load_gather(stg, [iota + k * L, iota + k * L])
            lm_acc += d16 * (m16 * wg)

    out_v[pl.ds(0, L)] = ss_acc
    out_v[pl.ds(L, L)] = lm_acc
    out_v[pl.ds(2 * L, L)] = g_acc
    out_v[pl.ds(3 * L, L)] = m_acc
    out_v[pl.ds(4 * L, L)] = nm_acc
    pltpu.sync_copy(out_v, out_hbm.at[wid])


_sc_call = functools.partial(
    pl.kernel,
    mesh=plsc.VectorSubcoreMesh(core_axis_name="c", subcore_axis_name="s"),
    compiler_params=pltpu.CompilerParams(needs_layout_passes=False),
    out_type=jax.ShapeDtypeStruct((NW, 5 * L), jnp.float32),
    scratch_types=[
        pltpu.VMEM((S, B), jnp.int32),
        pltpu.VMEM((S, B), jnp.float32),
        pltpu.VMEM((TCH, B), jnp.float32),
        pltpu.VMEM((TCH, B), jnp.float32),
        pltpu.VMEM((B, B), jnp.float32),
        pltpu.VMEM((B, B), jnp.float32),
        pltpu.VMEM((5 * L,), jnp.float32),
        pltpu.SemaphoreType.DMA,
        pltpu.SemaphoreType.DMA,
        pltpu.SemaphoreType.DMA,
        pltpu.SemaphoreType.DMA,
    ],
)(_sc_body)


@jax.jit
def kernel(inputs, rel_ress, targets, mask):
    inp_t = jnp.transpose(inputs, (1, 2, 0))
    rel_t = jnp.transpose(rel_ress, (1, 2, 0))
    t_t = jnp.transpose(targets.astype(jnp.int32), (1, 0))
    m_t = jnp.transpose(mask.astype(jnp.float32), (1, 0))
    out = _sc_call(inp_t, rel_t, t_t, m_t)
    s = out.reshape(NW, 5, L).sum(axis=(0, 2))
    lm_loss = -s[1] / s[3]
    rc_loss = (s[0] - 2.0 * s[2] + s[4]) / float(N * (NOUNS + 1))
    return lm_loss + rc_loss


# speedup vs baseline: 1.0273x; 1.0273x over previous
import functools

import jax
import jax.numpy as jnp
from jax import lax
from jax.experimental import pallas as pl
from jax.experimental.pallas import tpu as pltpu
from jax.experimental.pallas import tpu_sc as plsc

B, S, V, NOUNS = 128, 50, 9487, 1000
N = B * S
L = 16
NC, NS = 2, 16
NW = NC * NS
TCH = 200
RU = S * (NOUNS // TCH)
RIT = -(-RU // NW)
BCH = B // L


def _ge0(x):
    return 1 - lax.shift_right_logical(x, 31)


def _sc_body(inp_hbm, rel_hbm, t_hbm, m_hbm, out_hbm,
             t_v, m_v, bufr0, bufr1, stg0, stg1, out_v,
             semr0, semr1, semi0, semi1):
    wid = lax.axis_index("c") * NS + lax.axis_index("s")
    iota = lax.iota(jnp.int32, L)
    bufsr = (bufr0, bufr1)
    semsr = (semr0, semr1)

    def unit_su(u):
        u_cl = jnp.minimum(u, RU - 1)
        s = u_cl // 5
        t0 = pl.multiple_of((u_cl % 5) * TCH, 8)
        return s, t0, _ge0(RU - 1 - u).astype(jnp.float32)

    def start_r(u, slot):
        s, t0, _ = unit_su(u)
        return pltpu.async_copy(rel_hbm.at[s, pl.ds(t0, TCH), :],
                                bufsr[slot], semsr[slot])

    cpsr = [start_r(wid, 0), None]
    cpsr[1] = start_r(wid + NW, 1)
    pltpu.sync_copy(t_hbm, t_v)
    pltpu.sync_copy(m_hbm, m_v)

    lm_cps = []
    for srow_off, stg, sem in ((0, stg0, semi0), (NW, stg1, semi1)):
        s = jnp.minimum(wid + srow_off, S - 1)
        for k in range(BCH):
            t16 = jnp.clip(t_v[s, pl.ds(k * L, L)], 0, V - 1)
            lm_cps.append(pltpu.async_copy(
                inp_hbm.at[s].at[t16], stg.at[pl.ds(k * L, L), :], sem))

    m_acc = jnp.zeros((L,), jnp.float32)
    nm_acc = jnp.zeros((L,), jnp.float32)
    for srow_off in (0, NW):
        srow = wid + srow_off
        s = jnp.minimum(srow, S - 1)
        wg = _ge0(S - 1 - srow).astype(jnp.float32)
        for k in range(BCH):
            t16 = t_v[s, pl.ds(k * L, L)]
            m16 = m_v[s, pl.ds(k * L, L)]
            nm_acc += _ge0(NOUNS - t16).astype(jnp.float32) * wg
            m_acc += m16 * wg

    ss_acc = jnp.zeros((L,), jnp.float32)
    g_acc = jnp.zeros((L,), jnp.float32)
    for i in range(RIT):
        slot = i % 2
        u = wid + NW * i
        cpsr[slot].wait()
        buf = bufsr[slot]
        s, t0, wu = unit_su(u)

        def row(r, a, buf=buf):
            a0, a1, a2, a3 = a
            for rr in range(4):
                for k in range(0, BCH, 4):
                    x0 = buf[4 * r + rr, pl.ds(k * L, L)]
                    x1 = buf[4 * r + rr, pl.ds((k + 1) * L, L)]
                    x2 = buf[4 * r + rr, pl.ds((k + 2) * L, L)]
                    x3 = buf[4 * r + rr, pl.ds((k + 3) * L, L)]
                    a0 += x0 * x0
                    a1 += x1 * x1
                    a2 += x2 * x2
                    a3 += x3 * x3
            return (a0, a1, a2, a3)

        z = jnp.zeros((L,), jnp.float32)
        u0, u1, u2, u3 = lax.fori_loop(0, TCH // 4, row, (z, z, z, z))
        ss_acc += (u0 + u1 + u2 + u3) * wu

        for k in range(BCH):
            t16 = t_v[s, pl.ds(k * L, L)]
            tg = t16 - 1
            inr = (_ge0(tg - t0) * _ge0(t0 + (TCH - 1) - tg)).astype(jnp.float32)
            idx_t = jnp.clip(tg - t0, 0, TCH - 1)
            g_acc += plsc.load_gather(buf, [idx_t, iota + k * L]) * (inr * wu)

        if i + 2 < RIT:
            cpsr[slot] = start_r(wid + NW * (i + 2), slot)

    for cp in lm_cps:
        cp.wait()
    lm_acc = jnp.zeros((L,), jnp.float32)
    for srow_off, stg in ((0, stg0), (NW, stg1)):
        srow = wid + srow_off
        s = jnp.minimum(srow, S - 1)
        wg = _ge0(S - 1 - srow).astype(jnp.float32)
        for k in range(BCH):
            m16 = m_v[s, pl.ds(k * L, L)]
            d16 = plsc.load_gather(stg, [iota + k * L, iota + k * L])
            lm_acc += d16 * (m16 * wg)

    out_v[pl.ds(0, L)] = ss_acc
    out_v[pl.ds(L, L)] = lm_acc
    out_v[pl.ds(2 * L, L)] = g_acc
    out_v[pl.ds(3 * L, L)] = m_acc
    out_v[pl.ds(4 * L, L)] = nm_acc
    pltpu.sync_copy(out_v, out_hbm.at[wid])


_sc_call = functools.partial(
    pl.kernel,
    mesh=plsc.VectorSubcoreMesh(core_axis_name="c", subcore_axis_name="s"),
    compiler_params=pltpu.CompilerParams(needs_layout_passes=False),
    out_type=jax.ShapeDtypeStruct((NW, 5 * L), jnp.float32),
    scratch_types=[
        pltpu.VMEM((S, B), jnp.int32),
        pltpu.VMEM((S, B), jnp.float32),
        pltpu.VMEM((TCH, B), jnp.float32),
        pltpu.VMEM((TCH, B), jnp.float32),
        pltpu.VMEM((B, B), jnp.float32),
        pltpu.VMEM((B, B), jnp.float32),
        pltpu.VMEM((5 * L,), jnp.float32),
        pltpu.SemaphoreType.DMA,
        pltpu.SemaphoreType.DMA,
        pltpu.SemaphoreType.DMA,
        pltpu.SemaphoreType.DMA,
    ],
)(_sc_body)


@jax.jit
def kernel(inputs, rel_ress, targets, mask):
    inp_t = jnp.transpose(inputs, (1, 2, 0))
    rel_t = jnp.transpose(rel_ress, (1, 2, 0))
    t_t = jnp.transpose(targets.astype(jnp.int32), (1, 0))
    m_t = jnp.transpose(mask.astype(jnp.float32), (1, 0))
    out = _sc_call(inp_t, rel_t, t_t, m_t)
    s = out.reshape(NW, 5, L).sum(axis=(0, 2))
    lm_loss = -s[1] / s[3]
    rc_loss = (s[0] - 2.0 * s[2] + s[4]) / float(N * (NOUNS + 1))
    return lm_loss + rc_loss
